# Initial kernel scaffold; baseline (speedup 1.0000x reference)
import jax
import jax.numpy as jnp
from jax.experimental import pallas as pl


def kernel(x, gates, W_gate, b_gate, W_experts, b_experts):
    raise NotImplementedError("write your pallas kernel here")


# speedup vs baseline: 18.1748x; 18.1748x over previous
import jax
import jax.numpy as jnp
from jax.experimental import pallas as pl

_TILE = 512


def _moe_body(x_ref, gates_ref, wg_ref, bg_ref, wflat_ref, be_ref, out_ref):
    xb = x_ref[...]
    g = jnp.dot(xb, wg_ref[...], preferred_element_type=jnp.float32)
    g = (g + bg_ref[...]) * gates_ref[...]
    y = jnp.dot(xb, wflat_ref[...], preferred_element_type=jnp.float32)
    E, O = be_ref.shape
    acc = jnp.zeros((xb.shape[0], O), jnp.float32)
    for e in range(E):
        acc += g[:, e : e + 1] * (y[:, e * O : (e + 1) * O] + be_ref[e : e + 1, :])
    out_ref[...] = acc


def kernel(x, gates, W_gate, b_gate, W_experts, b_experts):
    B, D = x.shape
    E = gates.shape[1]
    O = W_experts.shape[2]
    w_flat = jnp.transpose(W_experts, (1, 0, 2)).reshape(D, E * O)
    bg2 = b_gate.reshape(1, E)
    tile = _TILE if B % _TILE == 0 else B
    grid = (B // tile,)
    return pl.pallas_call(
        _moe_body,
        grid=grid,
        in_specs=[
            pl.BlockSpec((tile, D), lambda i: (i, 0)),
            pl.BlockSpec((tile, E), lambda i: (i, 0)),
            pl.BlockSpec((D, E), lambda i: (0, 0)),
            pl.BlockSpec((1, E), lambda i: (0, 0)),
            pl.BlockSpec((D, E * O), lambda i: (0, 0)),
            pl.BlockSpec((E, O), lambda i: (0, 0)),
        ],
        out_specs=pl.BlockSpec((tile, O), lambda i: (i, 0)),
        out_shape=jax.ShapeDtypeStruct((B, O), jnp.float32),
    )(x, gates, W_gate, bg2, w_flat, b_experts)
